# Initial kernel scaffold; baseline (speedup 1.0000x reference)
import jax
import jax.numpy as jnp
from jax.experimental import pallas as pl


def kernel(x, table):
    raise NotImplementedError("write your pallas kernel here")


# speedup vs baseline: 3.4987x; 3.4987x over previous
import functools

import jax
import jax.numpy as jnp
from jax import lax
from jax.experimental import pallas as pl
from jax.experimental.pallas import tpu as pltpu
from jax.experimental.pallas import tpu_sc as plsc

EMBED_DIM = 256
CHUNK = 128


@functools.lru_cache(maxsize=None)
def _make_gather(num_rows, embed_dim):
    info = plsc.get_sparse_core_info()
    nc, ns = info.num_cores, info.num_subcores
    nw = nc * ns
    rows_per_w = num_rows // nw
    nchunk = rows_per_w // CHUNK
    assert rows_per_w * nw == num_rows and nchunk * CHUNK == rows_per_w
    mesh = plsc.VectorSubcoreMesh(core_axis_name="c", subcore_axis_name="s")

    @functools.partial(
        pl.kernel,
        mesh=mesh,
        out_type=jax.ShapeDtypeStruct((num_rows, embed_dim), jnp.float32),
        scratch_types=[
            pltpu.VMEM((nchunk, CHUNK), jnp.int32),
            pltpu.VMEM((CHUNK, embed_dim), jnp.float32),
            pltpu.VMEM((CHUNK, embed_dim), jnp.float32),
            pltpu.SemaphoreType.DMA,
            pltpu.SemaphoreType.DMA,
            pltpu.SemaphoreType.DMA,
            pltpu.SemaphoreType.DMA,
        ],
    )
    def gather_kernel(table_hbm, idx_hbm, out_hbm, idx_v, buf0, buf1,
                      g0, g1, o0, o1):
        wid = lax.axis_index("s") * nc + lax.axis_index("c")
        base = wid * rows_per_w
        pltpu.sync_copy(idx_hbm.at[wid], idx_v)

        bufs = (buf0, buf1)
        gsems = (g0, g1)
        osems = (o0, o1)

        def gather_copy(c, p):
            return pltpu.make_async_copy(
                table_hbm.at[idx_v.at[c]], bufs[p], gsems[p])

        def out_copy(c, p):
            return pltpu.make_async_copy(
                bufs[p], out_hbm.at[pl.ds(base + c * CHUNK, CHUNK)], osems[p])

        gather_copy(0, 0).start()
        gather_copy(0, 0).wait()
        gather_copy(1, 1).start()
        out_copy(0, 0).start()

        def body(i, _):
            c = 1 + 2 * i
            gather_copy(c, 1).wait()
            out_copy(c - 1, 0).wait()
            gather_copy(c + 1, 0).start()
            out_copy(c, 1).start()
            gather_copy(c + 1, 0).wait()
            out_copy(c, 1).wait()
            gather_copy(c + 2, 1).start()
            out_copy(c + 1, 0).start()
            return _

        lax.fori_loop(0, (nchunk - 2) // 2, body, None)

        last = nchunk - 1
        gather_copy(last, 1).wait()
        out_copy(last - 1, 0).wait()
        out_copy(last, 1).start()
        out_copy(last, 1).wait()

    return gather_kernel


def kernel(x, table):
    b, h = x.shape
    v, d = table.shape
    info = plsc.get_sparse_core_info()
    nw = info.num_cores * info.num_subcores
    num_rows = b * h
    nchunk = num_rows // (nw * CHUNK)
    idx3 = x.reshape(nw, nchunk, CHUNK).astype(jnp.int32)
    out = _make_gather(num_rows, d)(table, idx3)
    return out.reshape(b, h, d)


# speedup vs baseline: 3.5196x; 1.0060x over previous
import functools

import jax
import jax.numpy as jnp
from jax import lax
from jax.experimental import pallas as pl
from jax.experimental.pallas import tpu as pltpu
from jax.experimental.pallas import tpu_sc as plsc

EMBED_DIM = 256
CHUNK = 128
NBUF = 3


@functools.lru_cache(maxsize=None)
def _make_gather(num_rows, embed_dim):
    info = plsc.get_sparse_core_info()
    nc, ns = info.num_cores, info.num_subcores
    nw = nc * ns
    rows_per_w = num_rows // nw
    nchunk = rows_per_w // CHUNK
    assert rows_per_w * nw == num_rows and nchunk * CHUNK == rows_per_w
    assert nchunk >= 5 and (nchunk - 5) % NBUF == 0
    mesh = plsc.VectorSubcoreMesh(core_axis_name="c", subcore_axis_name="s")

    @functools.partial(
        pl.kernel,
        mesh=mesh,
        out_type=jax.ShapeDtypeStruct((num_rows, embed_dim), jnp.float32),
        scratch_types=[
            pltpu.VMEM((nchunk, CHUNK), jnp.int32),
            pltpu.VMEM((CHUNK, embed_dim), jnp.float32),
            pltpu.VMEM((CHUNK, embed_dim), jnp.float32),
            pltpu.VMEM((CHUNK, embed_dim), jnp.float32),
            pltpu.SemaphoreType.DMA,
            pltpu.SemaphoreType.DMA,
            pltpu.SemaphoreType.DMA,
            pltpu.SemaphoreType.DMA,
            pltpu.SemaphoreType.DMA,
            pltpu.SemaphoreType.DMA,
        ],
    )
    def gather_kernel(table_hbm, idx_hbm, out_hbm, idx_v, buf0, buf1, buf2,
                      g0, g1, g2, o0, o1, o2):
        wid = lax.axis_index("s") * nc + lax.axis_index("c")
        base = wid * rows_per_w
        pltpu.sync_copy(idx_hbm.at[wid], idx_v)

        bufs = (buf0, buf1, buf2)
        gsems = (g0, g1, g2)
        osems = (o0, o1, o2)

        def gather_copy(c, p):
            return pltpu.make_async_copy(
                table_hbm.at[idx_v.at[c]], bufs[p], gsems[p])

        def out_copy(c, p):
            return pltpu.make_async_copy(
                bufs[p], out_hbm.at[pl.ds(base + c * CHUNK, CHUNK)], osems[p])

        def steady(c, p):
            q = (p + 2) % NBUF
            gather_copy(c, p).wait()
            out_copy(c, p).start()
            out_copy(c - 1, q).wait()
            gather_copy(c + 2, q).start()

        gather_copy(0, 0).start()
        gather_copy(1, 1).start()
        gather_copy(0, 0).wait()
        out_copy(0, 0).start()
        gather_copy(2, 2).start()

        def body(i, _):
            c = 1 + NBUF * i
            steady(c, 1 % NBUF)
            steady(c + 1, 2 % NBUF)
            steady(c + 2, 0)
            return _

        lax.fori_loop(0, (nchunk - 5) // NBUF, body, None)

        m = nchunk - 4
        steady(m, m % NBUF)
        steady(m + 1, (m + 1) % NBUF)
        gather_copy(m + 2, (m + 2) % NBUF).wait()
        out_copy(m + 2, (m + 2) % NBUF).start()
        out_copy(m + 1, (m + 1) % NBUF).wait()
        gather_copy(m + 3, (m + 3) % NBUF).wait()
        out_copy(m + 3, (m + 3) % NBUF).start()
        out_copy(m + 2, (m + 2) % NBUF).wait()
        out_copy(m + 3, (m + 3) % NBUF).wait()

    return gather_kernel


def kernel(x, table):
    b, h = x.shape
    v, d = table.shape
    info = plsc.get_sparse_core_info()
    nw = info.num_cores * info.num_subcores
    num_rows = b * h
    nchunk = num_rows // (nw * CHUNK)
    idx3 = x.reshape(nw, nchunk, CHUNK).astype(jnp.int32)
    out = _make_gather(num_rows, d)(table, idx3)
    return out.reshape(b, h, d)
